# baseline (speedup 1.0000x reference)
import functools

import jax
import jax.numpy as jnp
from jax import lax
from jax.experimental import pallas as pl
from jax.experimental.pallas import tpu as pltpu
from jax.experimental.pallas import tpu_sc as plsc

B = 16384
EMB = 32
NROWS = 1_000_000
L = 16
NC = 2
NS = 16
NW = NC * NS
NTC = (NROWS + 127) // 128
CPW = 245
SPW = CPW * 128
CH = 8
UPC = CH * 128
NCH = -(-CPW // CH)
STG = 224
FLUSH_AT = STG - L
NPAD = 16
BPW = B // NW


def _gather_body(u_hbm, v_hbm, uet_hbm, iet_hbm, uvp_hbm,
                 idxbuf, mylist_v, mylist_b, chunkflat, stage2d, stageb,
                 sem_c, sem_f):
    wid = lax.axis_index("s") * NC + lax.axis_index("c")
    lanes = lax.iota(jnp.int32, L)
    lo = wid * SPW

    for t, (i_hbm, tab_hbm) in enumerate(((u_hbm, uet_hbm),
                                          (v_hbm, iet_hbm))):
        for r0 in range(0, STG, L):
            stageb[pl.ds(r0, L)] = B + (lanes + r0) % NPAD

        def compact_block(bi, cnt):
            def compact_group(i, cnt):
                vals = idxbuf[pl.ds(i * L, L)]
                bpos = bi * 1024 + i * L + lanes
                m = (vals >= lo) & (vals < lo + SPW)
                mi = m.astype(jnp.int32)
                pos = cnt + plsc.cumsum(mi) - 1
                plsc.store_scatter(mylist_v, [pos], vals, mask=m)
                plsc.store_scatter(mylist_b, [pos], bpos, mask=m)
                return cnt + jnp.sum(mi)
            pltpu.sync_copy(i_hbm.at[pl.ds(bi * 1024, 1024)], idxbuf)
            return lax.fori_loop(0, 1024 // L, compact_group, cnt)

        cnt = lax.fori_loop(0, B // 1024, compact_block, 0)
        ng = (cnt + L - 1) // L

        def chunk_body(k, spos):
            cb = wid * CPW + k * CH
            clo = cb * 128
            chi = jnp.minimum(clo + UPC, lo + SPW)
            ubase = jnp.minimum(cb, NTC - CH) * 128
            for q in range(4):
                for s in range(8):
                    pltpu.async_copy(
                        tab_hbm.at[q, s, pl.ds(ubase, UPC)],
                        chunkflat.at[pl.ds((q * 8 + s) * UPC, UPC)], sem_c)
            pltpu.make_async_copy(
                tab_hbm.at[0, 0, pl.ds(0, EMB * UPC)], chunkflat,
                sem_c).wait()

            def match_group(j, spos):
                vals = mylist_v[pl.ds(j * L, L)]
                bvals = mylist_b[pl.ds(j * L, L)]
                m = ((j * L + lanes) < cnt) & (vals >= clo) & (vals < chi)
                mi = m.astype(jnp.int32)
                msum = jnp.sum(mi)

                @pl.when(msum > 0)
                def _():
                    cols = jnp.where(m, vals - ubase, 0)
                    pos = spos + plsc.cumsum(mi) - 1
                    plsc.store_scatter(stageb, [pos], bvals, mask=m)
                    for e in range(EMB):
                        ve = plsc.load_gather(chunkflat, [cols + e * UPC])
                        plsc.store_scatter(
                            stage2d, [pos, jnp.full((L,), e, jnp.int32)],
                            ve, mask=m)

                spos2 = spos + msum

                @pl.when(spos2 >= FLUSH_AT)
                def _():
                    pltpu.async_copy(
                        stage2d, uvp_hbm.at[t].at[stageb], sem_f).wait()

                return jnp.where(spos2 >= FLUSH_AT, 0, spos2)

            return lax.fori_loop(0, ng, match_group, spos)

        spos = lax.fori_loop(0, NCH, chunk_body, 0)

        @pl.when(spos > 0)
        def _():
            pltpu.async_copy(stage2d, uvp_hbm.at[t].at[stageb], sem_f).wait()


def _dot_body(u_hbm, v_hbm, uvp_hbm, ub_hbm, ib_hbm, out_hbm,
              idx_u, idx_v, bu, bv, ru, rv, tr, out_v, sem):
    wid = lax.axis_index("s") * NC + lax.axis_index("c")
    base = wid * BPW
    lanes = lax.iota(jnp.int32, L)

    pltpu.sync_copy(u_hbm.at[pl.ds(base, BPW)], idx_u)
    pltpu.sync_copy(v_hbm.at[pl.ds(base, BPW)], idx_v)
    cbu = pltpu.async_copy(ub_hbm.at[idx_u], bu, sem)
    cbv = pltpu.async_copy(ib_hbm.at[idx_v], bv, sem)
    cbu.wait()
    cbv.wait()

    for blk in range(BPW // 128):
        pltpu.sync_copy(
            uvp_hbm.at[0, pl.ds(base + blk * 128, 128), :], ru)
        pltpu.sync_copy(
            uvp_hbm.at[1, pl.ds(base + blk * 128, 128), :], rv)
        for g in range(128 // L):
            for j in range(L):
                r = g * L + j
                t = (ru[r, pl.ds(0, L)] * rv[r, pl.ds(0, L)]
                     + ru[r, pl.ds(L, L)] * rv[r, pl.ds(L, L)])
                plsc.store_scatter(tr, [lanes * L + j], t)
            o = blk * 128 + g * L
            acc = bu[pl.ds(o, L)] + bv[pl.ds(o, L)]
            for l in range(L):
                acc = acc + tr[pl.ds(l * L, L)]
            out_v[pl.ds(o, L)] = acc

    pltpu.sync_copy(out_v, out_hbm.at[pl.ds(base, BPW)])


def kernel(u, v, user_emb, item_emb, user_bias, item_bias):
    mesh = plsc.VectorSubcoreMesh(core_axis_name="c", subcore_axis_name="s")

    k1 = functools.partial(
        pl.kernel,
        out_type=jax.ShapeDtypeStruct((2, B + NPAD, 128), jnp.float32),
        mesh=mesh,
        scratch_types=[
            pltpu.VMEM((1024,), jnp.int32),
            pltpu.VMEM((B + L,), jnp.int32),
            pltpu.VMEM((B + L,), jnp.int32),
            pltpu.VMEM((EMB * UPC,), jnp.float32),
            pltpu.VMEM((STG, 128), jnp.float32),
            pltpu.VMEM((STG,), jnp.int32),
            pltpu.SemaphoreType.DMA,
            pltpu.SemaphoreType.DMA,
        ],
        compiler_params=pltpu.CompilerParams(needs_layout_passes=False),
    )(_gather_body)

    k2 = functools.partial(
        pl.kernel,
        out_type=jax.ShapeDtypeStruct((B,), jnp.float32),
        mesh=mesh,
        scratch_types=[
            pltpu.VMEM((BPW,), jnp.int32),
            pltpu.VMEM((BPW,), jnp.int32),
            pltpu.VMEM((BPW,), jnp.float32),
            pltpu.VMEM((BPW,), jnp.float32),
            pltpu.VMEM((128, 128), jnp.float32),
            pltpu.VMEM((128, 128), jnp.float32),
            pltpu.VMEM((L * L,), jnp.float32),
            pltpu.VMEM((BPW,), jnp.float32),
            pltpu.SemaphoreType.DMA,
        ],
        compiler_params=pltpu.CompilerParams(
            needs_layout_passes=False, use_tc_tiling_on_sc=False),
    )(_dot_body)

    uet = user_emb.T.reshape(4, 8, NROWS)
    iet = item_emb.T.reshape(4, 8, NROWS)
    uvp = k1(u, v, uet, iet)
    return k2(u, v, uvp, user_bias.reshape(-1), item_bias.reshape(-1))


# speedup vs baseline: 1.3403x; 1.3403x over previous
import functools

import jax
import jax.numpy as jnp
from jax import lax
from jax.experimental import pallas as pl
from jax.experimental.pallas import tpu as pltpu
from jax.experimental.pallas import tpu_sc as plsc

B = 16384
EMB = 32
NROWS = 1_000_000
L = 16
NC = 2
NS = 16
NW = NC * NS
NTC = (NROWS + 127) // 128
CPW = 256
SPW = CPW * 128
CH = 8
UPC = CH * 128
NCH = CPW // CH
STG = 160
FLUSH_AT = STG - L
NPAD = 16
BPW = B // NW


def _gather_body(u_hbm, v_hbm, uet_hbm, iet_hbm, uvp_hbm,
                 idxbuf, mylist_v, mylist_b, bufa, bufb, stage2d, stageb,
                 sem_c, sem_f):
    wid = lax.axis_index("s") * NC + lax.axis_index("c")
    lanes = lax.iota(jnp.int32, L)
    lo = wid * SPW

    for t, (i_hbm, tab_hbm) in enumerate(((u_hbm, uet_hbm),
                                          (v_hbm, iet_hbm))):
        for r0 in range(0, STG, L):
            stageb[pl.ds(r0, L)] = B + (lanes + r0) % NPAD

        def compact_block(bi, cnt):
            def compact_group(i, cnt):
                vals = idxbuf[pl.ds(i * L, L)]
                bpos = bi * 1024 + i * L + lanes
                m = (vals >= lo) & (vals < lo + SPW)
                mi = m.astype(jnp.int32)
                pos = cnt + plsc.cumsum(mi) - 1
                plsc.store_scatter(mylist_v, [pos], vals, mask=m)
                plsc.store_scatter(mylist_b, [pos], bpos, mask=m)
                return cnt + jnp.sum(mi)
            pltpu.sync_copy(i_hbm.at[pl.ds(bi * 1024, 1024)], idxbuf)
            return lax.fori_loop(0, 1024 // L, compact_group, cnt)

        cnt = lax.fori_loop(0, B // 1024, compact_block, 0)
        ng = (cnt + L - 1) // L

        def chunk_lo(k):
            return (wid * CPW + k * CH) * 128

        def issue(k, buf):
            @pl.when(chunk_lo(k) < NROWS)
            def _():
                ubase = jnp.minimum(wid * CPW + k * CH, NTC - CH) * 128
                for q in range(4):
                    pltpu.async_copy(
                        tab_hbm.at[q, :, pl.ds(ubase, UPC)],
                        buf.at[pl.ds(q * 8, 8), :], sem_c)

        def drain(k, buf):
            @pl.when(chunk_lo(k) < NROWS)
            def _():
                for q in range(4):
                    pltpu.make_async_copy(
                        tab_hbm.at[q, :, pl.ds(0, UPC)],
                        buf.at[pl.ds(q * 8, 8), :], sem_c).wait()

        def extract(k, buf, spos):
            clo = chunk_lo(k)
            ubase = jnp.minimum(wid * CPW + k * CH, NTC - CH) * 128

            def match_group(j, spos):
                vals = mylist_v[pl.ds(j * L, L)]
                bvals = mylist_b[pl.ds(j * L, L)]
                m = ((j * L + lanes) < cnt) & (vals >= clo) & (vals < clo + UPC)
                mi = m.astype(jnp.int32)
                msum = jnp.sum(mi)

                @pl.when(msum > 0)
                def _():
                    cols = jnp.where(m, vals - ubase, 0)
                    pos = spos + plsc.cumsum(mi) - 1
                    plsc.store_scatter(stageb, [pos], bvals, mask=m)
                    for e in range(EMB):
                        ve = plsc.load_gather(
                            buf, [jnp.full((L,), e, jnp.int32), cols])
                        plsc.store_scatter(
                            stage2d, [pos, jnp.full((L,), e, jnp.int32)],
                            ve, mask=m)

                spos2 = spos + msum

                @pl.when(spos2 >= FLUSH_AT)
                def _():
                    pltpu.async_copy(
                        stage2d, uvp_hbm.at[t].at[stageb], sem_f).wait()

                return jnp.where(spos2 >= FLUSH_AT, 0, spos2)

            return lax.fori_loop(0, ng, match_group, spos)

        issue(0, bufa)

        def pair_body(mm, spos):
            k0 = mm * 2
            drain(k0, bufa)
            issue(k0 + 1, bufb)
            spos = extract(k0, bufa, spos)
            drain(k0 + 1, bufb)

            @pl.when(k0 + 2 < NCH)
            def _():
                issue(k0 + 2, bufa)

            return extract(k0 + 1, bufb, spos)

        spos = lax.fori_loop(0, NCH // 2, pair_body, 0)

        @pl.when(spos > 0)
        def _():
            pltpu.async_copy(stage2d, uvp_hbm.at[t].at[stageb], sem_f).wait()


def _dot_body(u_hbm, v_hbm, uvp_hbm, ub_hbm, ib_hbm, out_hbm,
              idx_u, idx_v, bu, bv, ru, rv, tr, out_v, sem):
    wid = lax.axis_index("s") * NC + lax.axis_index("c")
    base = wid * BPW
    lanes = lax.iota(jnp.int32, L)

    pltpu.sync_copy(u_hbm.at[pl.ds(base, BPW)], idx_u)
    pltpu.sync_copy(v_hbm.at[pl.ds(base, BPW)], idx_v)
    cbu = pltpu.async_copy(ub_hbm.at[idx_u], bu, sem)
    cbv = pltpu.async_copy(ib_hbm.at[idx_v], bv, sem)
    cbu.wait()
    cbv.wait()

    for blk in range(BPW // 128):
        pltpu.sync_copy(
            uvp_hbm.at[0, pl.ds(base + blk * 128, 128), :], ru)
        pltpu.sync_copy(
            uvp_hbm.at[1, pl.ds(base + blk * 128, 128), :], rv)
        for g in range(128 // L):
            for j in range(L):
                r = g * L + j
                t = (ru[r, pl.ds(0, L)] * rv[r, pl.ds(0, L)]
                     + ru[r, pl.ds(L, L)] * rv[r, pl.ds(L, L)])
                plsc.store_scatter(tr, [lanes * L + j], t)
            o = blk * 128 + g * L
            acc = bu[pl.ds(o, L)] + bv[pl.ds(o, L)]
            for l in range(L):
                acc = acc + tr[pl.ds(l * L, L)]
            out_v[pl.ds(o, L)] = acc

    pltpu.sync_copy(out_v, out_hbm.at[pl.ds(base, BPW)])


def kernel(u, v, user_emb, item_emb, user_bias, item_bias):
    mesh = plsc.VectorSubcoreMesh(core_axis_name="c", subcore_axis_name="s")

    k1 = functools.partial(
        pl.kernel,
        out_type=jax.ShapeDtypeStruct((2, B + NPAD, 128), jnp.float32),
        mesh=mesh,
        scratch_types=[
            pltpu.VMEM((1024,), jnp.int32),
            pltpu.VMEM((B + L,), jnp.int32),
            pltpu.VMEM((B + L,), jnp.int32),
            pltpu.VMEM((EMB, UPC), jnp.float32),
            pltpu.VMEM((EMB, UPC), jnp.float32),
            pltpu.VMEM((STG, 128), jnp.float32),
            pltpu.VMEM((STG,), jnp.int32),
            pltpu.SemaphoreType.DMA,
            pltpu.SemaphoreType.DMA,
        ],
        compiler_params=pltpu.CompilerParams(needs_layout_passes=False),
    )(_gather_body)

    k2 = functools.partial(
        pl.kernel,
        out_type=jax.ShapeDtypeStruct((B,), jnp.float32),
        mesh=mesh,
        scratch_types=[
            pltpu.VMEM((BPW,), jnp.int32),
            pltpu.VMEM((BPW,), jnp.int32),
            pltpu.VMEM((BPW,), jnp.float32),
            pltpu.VMEM((BPW,), jnp.float32),
            pltpu.VMEM((128, 128), jnp.float32),
            pltpu.VMEM((128, 128), jnp.float32),
            pltpu.VMEM((L * L,), jnp.float32),
            pltpu.VMEM((BPW,), jnp.float32),
            pltpu.SemaphoreType.DMA,
        ],
        compiler_params=pltpu.CompilerParams(
            needs_layout_passes=False, use_tc_tiling_on_sc=False),
    )(_dot_body)

    uet = user_emb.T.reshape(4, 8, NROWS)
    iet = item_emb.T.reshape(4, 8, NROWS)
    uvp = k1(u, v, uet, iet)
    return k2(u, v, uvp, user_bias.reshape(-1), item_bias.reshape(-1))


# speedup vs baseline: 1.3551x; 1.0111x over previous
import functools

import jax
import jax.numpy as jnp
from jax import lax
from jax.experimental import pallas as pl
from jax.experimental.pallas import tpu as pltpu
from jax.experimental.pallas import tpu_sc as plsc

B = 16384
EMB = 32
NROWS = 1_000_000
L = 16
NC = 2
NS = 16
NW = NC * NS
NTC = (NROWS + 127) // 128
CPW = 256
SPW = CPW * 128
CH = 8
UPC = CH * 128
NCH = CPW // CH
STG = 160
FLUSH_AT = STG - L
NPAD = 16
BPW = B // NW


def _gather_body(u_hbm, v_hbm, uet_hbm, iet_hbm, uvp_hbm,
                 idxbuf, mylist_v, mylist_b, bufa, bufb, stage2d, stageb,
                 sem_c, sem_f):
    wid = lax.axis_index("s") * NC + lax.axis_index("c")
    lanes = lax.iota(jnp.int32, L)
    lo = wid * SPW

    for t, (i_hbm, tab_hbm) in enumerate(((u_hbm, uet_hbm),
                                          (v_hbm, iet_hbm))):
        for r0 in range(0, STG, L):
            stageb[pl.ds(r0, L)] = B + (lanes + r0) % NPAD

        def compact_block(bi, cnt):
            def compact_group(i, cnt):
                vals = idxbuf[pl.ds(i * L, L)]
                bpos = bi * 1024 + i * L + lanes
                m = (vals >= lo) & (vals < lo + SPW)
                mi = m.astype(jnp.int32)
                pos = cnt + plsc.cumsum(mi) - 1
                plsc.store_scatter(mylist_v, [pos], vals, mask=m)
                plsc.store_scatter(mylist_b, [pos], bpos, mask=m)
                return cnt + jnp.sum(mi)
            pltpu.sync_copy(i_hbm.at[pl.ds(bi * 1024, 1024)], idxbuf)
            return lax.fori_loop(0, 1024 // L, compact_group, cnt)

        cnt = lax.fori_loop(0, B // 1024, compact_block, 0)
        ng = (cnt + L - 1) // L

        def chunk_lo(k):
            return (wid * CPW + k * CH) * 128

        def issue(k, buf):
            @pl.when(chunk_lo(k) < NROWS)
            def _():
                ubase = jnp.minimum(wid * CPW + k * CH, NTC - CH) * 128
                pltpu.async_copy(
                    tab_hbm.at[:, :, pl.ds(ubase, UPC)], buf, sem_c)

        def drain(k, buf):
            @pl.when(chunk_lo(k) < NROWS)
            def _():
                pltpu.make_async_copy(
                    tab_hbm.at[:, :, pl.ds(0, UPC)], buf, sem_c).wait()

        def extract(k, buf, spos):
            clo = chunk_lo(k)
            ubase = jnp.minimum(wid * CPW + k * CH, NTC - CH) * 128

            def match_group(j, spos):
                vals = mylist_v[pl.ds(j * L, L)]
                bvals = mylist_b[pl.ds(j * L, L)]
                m = ((j * L + lanes) < cnt) & (vals >= clo) & (vals < clo + UPC)
                mi = m.astype(jnp.int32)
                msum = jnp.sum(mi)

                @pl.when(msum > 0)
                def _():
                    cols = jnp.where(m, vals - ubase, 0)
                    pos = spos + plsc.cumsum(mi) - 1
                    plsc.store_scatter(stageb, [pos], bvals, mask=m)
                    for e in range(EMB):
                        ve = plsc.load_gather(
                            buf, [jnp.full((L,), e // 8, jnp.int32),
                                  jnp.full((L,), e % 8, jnp.int32), cols])
                        plsc.store_scatter(
                            stage2d, [pos, jnp.full((L,), e, jnp.int32)],
                            ve, mask=m)

                spos2 = spos + msum

                @pl.when(spos2 >= FLUSH_AT)
                def _():
                    pltpu.async_copy(
                        stage2d, uvp_hbm.at[t].at[stageb], sem_f).wait()

                return jnp.where(spos2 >= FLUSH_AT, 0, spos2)

            return lax.fori_loop(0, ng, match_group, spos)

        issue(0, bufa)

        def pair_body(mm, spos):
            k0 = mm * 2
            drain(k0, bufa)
            issue(k0 + 1, bufb)
            spos = extract(k0, bufa, spos)
            drain(k0 + 1, bufb)

            @pl.when(k0 + 2 < NCH)
            def _():
                issue(k0 + 2, bufa)

            return extract(k0 + 1, bufb, spos)

        spos = lax.fori_loop(0, NCH // 2, pair_body, 0)

        @pl.when(spos > 0)
        def _():
            pltpu.async_copy(stage2d, uvp_hbm.at[t].at[stageb], sem_f).wait()


def _dot_body(u_hbm, v_hbm, uvp_hbm, ub_hbm, ib_hbm, out_hbm,
              idx_u, idx_v, bu, bv, ru, rv, tr, out_v, sem):
    wid = lax.axis_index("s") * NC + lax.axis_index("c")
    base = wid * BPW
    lanes = lax.iota(jnp.int32, L)

    pltpu.sync_copy(u_hbm.at[pl.ds(base, BPW)], idx_u)
    pltpu.sync_copy(v_hbm.at[pl.ds(base, BPW)], idx_v)
    cbu = pltpu.async_copy(ub_hbm.at[idx_u], bu, sem)
    cbv = pltpu.async_copy(ib_hbm.at[idx_v], bv, sem)
    cbu.wait()
    cbv.wait()

    for blk in range(BPW // 128):
        pltpu.sync_copy(
            uvp_hbm.at[0, pl.ds(base + blk * 128, 128), :], ru)
        pltpu.sync_copy(
            uvp_hbm.at[1, pl.ds(base + blk * 128, 128), :], rv)
        for g in range(128 // L):
            for j in range(L):
                r = g * L + j
                t = (ru[r, pl.ds(0, L)] * rv[r, pl.ds(0, L)]
                     + ru[r, pl.ds(L, L)] * rv[r, pl.ds(L, L)])
                plsc.store_scatter(tr, [lanes * L + j], t)
            o = blk * 128 + g * L
            acc = bu[pl.ds(o, L)] + bv[pl.ds(o, L)]
            for l in range(L):
                acc = acc + tr[pl.ds(l * L, L)]
            out_v[pl.ds(o, L)] = acc

    pltpu.sync_copy(out_v, out_hbm.at[pl.ds(base, BPW)])


def kernel(u, v, user_emb, item_emb, user_bias, item_bias):
    mesh = plsc.VectorSubcoreMesh(core_axis_name="c", subcore_axis_name="s")

    k1 = functools.partial(
        pl.kernel,
        out_type=jax.ShapeDtypeStruct((2, B + NPAD, 128), jnp.float32),
        mesh=mesh,
        scratch_types=[
            pltpu.VMEM((1024,), jnp.int32),
            pltpu.VMEM((B + L,), jnp.int32),
            pltpu.VMEM((B + L,), jnp.int32),
            pltpu.VMEM((4, 8, UPC), jnp.float32),
            pltpu.VMEM((4, 8, UPC), jnp.float32),
            pltpu.VMEM((STG, 128), jnp.float32),
            pltpu.VMEM((STG,), jnp.int32),
            pltpu.SemaphoreType.DMA,
            pltpu.SemaphoreType.DMA,
        ],
        compiler_params=pltpu.CompilerParams(needs_layout_passes=False),
    )(_gather_body)

    k2 = functools.partial(
        pl.kernel,
        out_type=jax.ShapeDtypeStruct((B,), jnp.float32),
        mesh=mesh,
        scratch_types=[
            pltpu.VMEM((BPW,), jnp.int32),
            pltpu.VMEM((BPW,), jnp.int32),
            pltpu.VMEM((BPW,), jnp.float32),
            pltpu.VMEM((BPW,), jnp.float32),
            pltpu.VMEM((128, 128), jnp.float32),
            pltpu.VMEM((128, 128), jnp.float32),
            pltpu.VMEM((L * L,), jnp.float32),
            pltpu.VMEM((BPW,), jnp.float32),
            pltpu.SemaphoreType.DMA,
        ],
        compiler_params=pltpu.CompilerParams(
            needs_layout_passes=False, use_tc_tiling_on_sc=False),
    )(_dot_body)

    uet = user_emb.T.reshape(4, 8, NROWS)
    iet = item_emb.T.reshape(4, 8, NROWS)
    uvp = k1(u, v, uet, iet)
    return k2(u, v, uvp, user_bias.reshape(-1), item_bias.reshape(-1))
